# Initial kernel scaffold; baseline (speedup 1.0000x reference)
import functools

import jax
import jax.numpy as jnp
from jax import lax
from jax.experimental import pallas as pl
from jax.experimental.pallas import tpu as pltpu
from jax.experimental.pallas import tpu_sc as plsc

N_NODES = 200
BINS = 50
LAST = 1.0
P = N_NODES * (N_NODES - 1) // 2
C = P * BINS
T_EVENTS = 1000000
T_TOTAL = T_EVENTS + C

_NC, _NS, _L = 2, 16, 16
_NW = _NC * _NS

_B = 2048
_NCHUNK = 992
_TPAD = _NCHUNK * _B

_NB_ROWS = 8192
_NB_PAD = _NB_ROWS * 128


def _nb_body(bl_ref, x_ref, o_ref):
    x = x_ref[...]
    acc = jnp.zeros(x.shape, jnp.int32)
    for k in range(BINS):
        acc += (x >= bl_ref[0, k]).astype(jnp.int32)
    o_ref[...] = acc


def _num_borders_le(ts, bl):
    n = ts.shape[0]
    tpad = jnp.full((_NB_PAD,), 2.0, jnp.float32).at[:n].set(ts)
    tpad = tpad.reshape(_NB_ROWS, 128)
    blp = jnp.zeros((1, 64), jnp.float32).at[0, :BINS].set(bl)
    grid = 8
    rows = _NB_ROWS // grid
    out = pl.pallas_call(
        _nb_body,
        grid=(grid,),
        in_specs=[
            pl.BlockSpec(memory_space=pltpu.SMEM),
            pl.BlockSpec((rows, 128), lambda g: (g, 0)),
        ],
        out_specs=pl.BlockSpec((rows, 128), lambda g: (g, 0)),
        out_shape=jax.ShapeDtypeStruct((_NB_ROWS, 128), jnp.int32),
    )(blp, tpad)
    return out.reshape(-1)[:n]


def _assemble_sc(t_cat, state_cat, delta_cat, starts_pad):
    mesh = plsc.VectorSubcoreMesh(core_axis_name="c", subcore_axis_name="s")
    out_type = (
        jax.ShapeDtypeStruct((_TPAD,), jnp.float32),
        jax.ShapeDtypeStruct((_TPAD,), jnp.int32),
        jax.ShapeDtypeStruct((_TPAD,), jnp.int32),
        jax.ShapeDtypeStruct((_TPAD,), jnp.float32),
    )
    scratch = [
        pltpu.VMEM((128,), jnp.int32),
        pltpu.VMEM((_B,), jnp.int32),
        pltpu.VMEM((_B,), jnp.int32),
        pltpu.VMEM((_B,), jnp.float32),
        pltpu.VMEM((_B,), jnp.int32),
        pltpu.VMEM((_B,), jnp.float32),
        pltpu.SemaphoreType.DMA,
        pltpu.SemaphoreType.DMA,
        pltpu.SemaphoreType.DMA,
    ]

    @functools.partial(pl.kernel, mesh=mesh, out_type=out_type,
                       scratch_types=scratch)
    def k(tc_hbm, sc_hbm, dc_hbm, st_hbm, t_out, ev_out, s_out, d_out,
          starts_v, idx_v, ev_v, tg_v, sg_v, dg_v, sem1, sem2, sem3):
        wid = lax.axis_index("s") * _NC + lax.axis_index("c")
        pltpu.sync_copy(st_hbm, starts_v)
        lane = lax.iota(jnp.int32, _L)

        def chunk_body(tt, carry):
            base = (wid + tt * _NW) * _B

            def vec_body(vi, c2):
                q = base + vi * _L + lane
                pos = jnp.zeros((_L,), jnp.int32)
                for s in (64, 32, 16, 8, 4, 2, 1):
                    cand = pos + s
                    sv = plsc.load_gather(starts_v, [cand])
                    pos = jnp.where(sv <= q, cand, pos)
                sstart = plsc.load_gather(starts_v, [pos])
                kreg = lax.shift_right_logical(pos, 1)
                is_bd = (pos & 1) == 0
                idx_bd = (q - sstart) * BINS + kreg
                idx_ev = C + q - (kreg + 1) * P
                idx = jnp.where(is_bd, idx_bd, idx_ev)
                idx = jnp.minimum(jnp.maximum(idx, 0), T_TOTAL - 1)
                idx_v[pl.ds(vi * _L, _L)] = idx
                ev_v[pl.ds(vi * _L, _L)] = jnp.where(
                    is_bd, jnp.zeros((_L,), jnp.int32),
                    jnp.ones((_L,), jnp.int32))
                return c2

            lax.fori_loop(0, _B // _L, vec_body, 0)
            cp1 = pltpu.async_copy(tc_hbm.at[idx_v], tg_v, sem1)
            cp2 = pltpu.async_copy(sc_hbm.at[idx_v], sg_v, sem2)
            cp3 = pltpu.async_copy(dc_hbm.at[idx_v], dg_v, sem3)
            cp1.wait()
            cp2.wait()
            cp3.wait()
            pltpu.sync_copy(tg_v, t_out.at[pl.ds(base, _B)])
            pltpu.sync_copy(ev_v, ev_out.at[pl.ds(base, _B)])
            pltpu.sync_copy(sg_v, s_out.at[pl.ds(base, _B)])
            pltpu.sync_copy(dg_v, d_out.at[pl.ds(base, _B)])
            return carry

        lax.fori_loop(0, _NCHUNK // _NW, chunk_body, 0)

    return k(t_cat, state_cat, delta_cat, starts_pad)


def kernel(pairs, times):
    n = N_NODES
    i = pairs[0].astype(jnp.int32)
    j = pairs[1].astype(jnp.int32)
    rows = i * (2 * n - i - 1) // 2 + (j - i - 1)
    bl = jnp.linspace(0.0, LAST, BINS + 1)[:-1].astype(jnp.float32)
    nev = times.shape[0]

    ts, row_s = lax.sort((times, rows), num_keys=1, is_stable=True)
    nb_s = _num_borders_le(ts, bl)
    e_cnt = jnp.searchsorted(ts, bl, side='left').astype(jnp.int32)

    row_g, e_g = lax.sort(
        (row_s, jnp.arange(nev, dtype=jnp.int32)), num_keys=1, is_stable=True)
    t_g = ts[e_g]
    bin_g = nb_s[e_g] - 1
    kq = row_g * BINS + bin_g
    g_idx = jnp.arange(nev, dtype=jnp.int32)
    nxt_kq = jnp.concatenate([kq[1:], jnp.full((1,), -1, jnp.int32)])
    is_end = kq != nxt_kq
    is_start = jnp.concatenate([jnp.array([True]), kq[1:] != kq[:-1]])
    cum_raw = jnp.zeros((C,), jnp.int32).at[
        jnp.where(is_end, kq, C)].set(g_idx + 1, mode='drop')
    cum = lax.cummax(cum_raw)
    m_first = jnp.zeros((C,), jnp.float32).at[
        jnp.where(is_start, kq, C)].set(t_g, mode='drop')
    has = cum_raw > 0
    cum_pad = jnp.concatenate([jnp.zeros((1,), jnp.int32), cum[:-1]])

    cum_m = cum_pad.reshape(P, BINS)
    state_bd = ((cum_m - cum_m[:, :1]) & 1).astype(jnp.int32)
    blnext = jnp.concatenate([bl[1:], jnp.full((1,), LAST, jnp.float32)])
    delta_bd = jnp.where(has.reshape(P, BINS), m_first.reshape(P, BINS),
                         blnext[None, :]) - bl[None, :]
    t_bd = jnp.tile(bl, P)

    row_start = cum_pad[row_g * BINS]
    state_g = ((g_idx - row_start + 1) & 1).astype(jnp.int32)
    nxt_row = jnp.concatenate([row_g[1:], jnp.full((1,), -1, jnp.int32)])
    nxt_bin = jnp.concatenate([bin_g[1:], jnp.full((1,), -1, jnp.int32)])
    nxt_t = jnp.concatenate([t_g[1:], jnp.zeros((1,), jnp.float32)])
    same = (nxt_row == row_g) & (nxt_bin == bin_g)
    delta_g = jnp.where(same, nxt_t, blnext[bin_g]) - t_g
    state_ev = jnp.zeros((nev,), jnp.int32).at[e_g].set(state_g)
    delta_ev = jnp.zeros((nev,), jnp.float32).at[e_g].set(delta_g)

    t_cat = jnp.concatenate([t_bd, ts])
    state_cat = jnp.concatenate([state_bd.reshape(-1), state_ev])
    delta_cat = jnp.concatenate([delta_bd.reshape(-1), delta_ev])
    k_arr = jnp.arange(BINS, dtype=jnp.int32)
    bstart = k_arr * P + e_cnt
    estart = (k_arr + 1) * P + e_cnt
    starts = jnp.stack([bstart, estart], axis=1).reshape(-1)
    starts_pad = jnp.full((128,), _TPAD, jnp.int32).at[:2 * BINS].set(starts)

    t_o, ev_o, s_o, d_o = _assemble_sc(t_cat, state_cat, delta_cat, starts_pad)
    return (t_o[:T_TOTAL], ev_o[:T_TOTAL].astype(bool), s_o[:T_TOTAL],
            d_o[:T_TOTAL])


# speedup vs baseline: 2.3092x; 2.3092x over previous
import functools

import jax
import jax.numpy as jnp
from jax import lax
from jax.experimental import pallas as pl
from jax.experimental.pallas import tpu as pltpu
from jax.experimental.pallas import tpu_sc as plsc

N_NODES = 200
BINS = 50
LAST = 1.0
P = N_NODES * (N_NODES - 1) // 2
C = P * BINS
T_EVENTS = 1000000
T_TOTAL = T_EVENTS + C

_NC, _NS, _L = 2, 16, 16
_NW = _NC * _NS

_B = 2048
_NCHUNK = 992
_TPAD = _NCHUNK * _B

_NB_ROWS = 8192
_NB_PAD = _NB_ROWS * 128


def _nb_body(bl_ref, x_ref, o_ref):
    x = x_ref[...]
    acc = jnp.zeros(x.shape, jnp.int32)
    for k in range(BINS):
        acc += (x >= bl_ref[0, k]).astype(jnp.int32)
    o_ref[...] = acc


def _num_borders_le(ts, bl):
    n = ts.shape[0]
    tpad = jnp.full((_NB_PAD,), 2.0, jnp.float32).at[:n].set(ts)
    tpad = tpad.reshape(_NB_ROWS, 128)
    blp = jnp.zeros((1, 64), jnp.float32).at[0, :BINS].set(bl)
    grid = 8
    rows = _NB_ROWS // grid
    out = pl.pallas_call(
        _nb_body,
        grid=(grid,),
        in_specs=[
            pl.BlockSpec(memory_space=pltpu.SMEM),
            pl.BlockSpec((rows, 128), lambda g: (g, 0)),
        ],
        out_specs=pl.BlockSpec((rows, 128), lambda g: (g, 0)),
        out_shape=jax.ShapeDtypeStruct((_NB_ROWS, 128), jnp.int32),
    )(blp, tpad)
    return out.reshape(-1)[:n]


def _assemble_sc(t_cat, state_cat, delta_cat, starts_pad):
    mesh = plsc.VectorSubcoreMesh(core_axis_name="c", subcore_axis_name="s")
    out_type = (
        jax.ShapeDtypeStruct((_TPAD,), jnp.float32),
        jax.ShapeDtypeStruct((_TPAD,), jnp.int32),
        jax.ShapeDtypeStruct((_TPAD,), jnp.int32),
        jax.ShapeDtypeStruct((_TPAD,), jnp.float32),
    )
    scratch = [
        pltpu.VMEM((128,), jnp.int32),
        pltpu.VMEM((_B,), jnp.int32),
        pltpu.VMEM((_B,), jnp.int32),
        pltpu.VMEM((_B,), jnp.float32),
        pltpu.VMEM((_B,), jnp.int32),
        pltpu.VMEM((_B,), jnp.float32),
        pltpu.SemaphoreType.DMA,
        pltpu.SemaphoreType.DMA,
        pltpu.SemaphoreType.DMA,
    ]

    @functools.partial(
        pl.kernel, mesh=mesh, out_type=out_type, scratch_types=scratch,
        compiler_params=pltpu.CompilerParams(needs_layout_passes=False))
    def k(tc_hbm, sc_hbm, dc_hbm, st_hbm, t_out, ev_out, s_out, d_out,
          starts_v, idx_v, ev_v, tg_v, sg_v, dg_v, sem1, sem2, sem3):
        wid = lax.axis_index("s") * _NC + lax.axis_index("c")
        pltpu.sync_copy(st_hbm, starts_v)
        lane = lax.iota(jnp.int32, _L)

        def chunk_body(tt, carry):
            base = (wid + tt * _NW) * _B

            def vec_body(vi, c2):
                q = base + vi * _L + lane
                pos = jnp.zeros((_L,), jnp.int32)
                for s in (64, 32, 16, 8, 4, 2, 1):
                    cand = pos + s
                    sv = plsc.load_gather(starts_v, [cand])
                    pos = jnp.where(sv <= q, cand, pos)
                sstart = plsc.load_gather(starts_v, [pos])
                kreg = lax.shift_right_logical(pos, 1)
                is_bd = (pos & 1) == 0
                idx_bd = (q - sstart) * BINS + kreg
                idx_ev = C + q - (kreg + 1) * P
                idx = jnp.where(is_bd, idx_bd, idx_ev)
                idx = jnp.minimum(jnp.maximum(idx, 0), T_TOTAL - 1)
                idx_v[pl.ds(vi * _L, _L)] = idx
                ev_v[pl.ds(vi * _L, _L)] = jnp.where(
                    is_bd, jnp.zeros((_L,), jnp.int32),
                    jnp.ones((_L,), jnp.int32))
                return c2

            lax.fori_loop(0, _B // _L, vec_body, 0)
            cp1 = pltpu.async_copy(tc_hbm.at[idx_v], tg_v, sem1)
            cp2 = pltpu.async_copy(sc_hbm.at[idx_v], sg_v, sem2)
            cp3 = pltpu.async_copy(dc_hbm.at[idx_v], dg_v, sem3)
            cp1.wait()
            cp2.wait()
            cp3.wait()
            pltpu.sync_copy(tg_v, t_out.at[pl.ds(base, _B)])
            pltpu.sync_copy(ev_v, ev_out.at[pl.ds(base, _B)])
            pltpu.sync_copy(sg_v, s_out.at[pl.ds(base, _B)])
            pltpu.sync_copy(dg_v, d_out.at[pl.ds(base, _B)])
            return carry

        lax.fori_loop(0, _NCHUNK // _NW, chunk_body, 0)

    return k(t_cat, state_cat, delta_cat, starts_pad)


def kernel(pairs, times):
    n = N_NODES
    i = pairs[0].astype(jnp.int32)
    j = pairs[1].astype(jnp.int32)
    rows = i * (2 * n - i - 1) // 2 + (j - i - 1)
    bl = jnp.linspace(0.0, LAST, BINS + 1)[:-1].astype(jnp.float32)
    nev = times.shape[0]

    ts, row_s = lax.sort((times, rows), num_keys=1, is_stable=True)
    nb_s = _num_borders_le(ts, bl)
    e_cnt = jnp.searchsorted(ts, bl, side='left').astype(jnp.int32)

    row_g, e_g = lax.sort(
        (row_s, jnp.arange(nev, dtype=jnp.int32)), num_keys=1, is_stable=True)
    t_g = ts[e_g]
    bin_g = nb_s[e_g] - 1
    kq = row_g * BINS + bin_g
    g_idx = jnp.arange(nev, dtype=jnp.int32)
    nxt_kq = jnp.concatenate([kq[1:], jnp.full((1,), -1, jnp.int32)])
    is_end = kq != nxt_kq
    is_start = jnp.concatenate([jnp.array([True]), kq[1:] != kq[:-1]])
    cum_raw = jnp.zeros((C,), jnp.int32).at[
        jnp.where(is_end, kq, C)].set(g_idx + 1, mode='drop')
    cum = lax.cummax(cum_raw)
    m_first = jnp.zeros((C,), jnp.float32).at[
        jnp.where(is_start, kq, C)].set(t_g, mode='drop')
    has = cum_raw > 0
    cum_pad = jnp.concatenate([jnp.zeros((1,), jnp.int32), cum[:-1]])

    cum_m = cum_pad.reshape(P, BINS)
    state_bd = ((cum_m - cum_m[:, :1]) & 1).astype(jnp.int32)
    blnext = jnp.concatenate([bl[1:], jnp.full((1,), LAST, jnp.float32)])
    delta_bd = jnp.where(has.reshape(P, BINS), m_first.reshape(P, BINS),
                         blnext[None, :]) - bl[None, :]
    t_bd = jnp.tile(bl, P)

    row_start = cum_pad[row_g * BINS]
    state_g = ((g_idx - row_start + 1) & 1).astype(jnp.int32)
    nxt_row = jnp.concatenate([row_g[1:], jnp.full((1,), -1, jnp.int32)])
    nxt_bin = jnp.concatenate([bin_g[1:], jnp.full((1,), -1, jnp.int32)])
    nxt_t = jnp.concatenate([t_g[1:], jnp.zeros((1,), jnp.float32)])
    same = (nxt_row == row_g) & (nxt_bin == bin_g)
    delta_g = jnp.where(same, nxt_t, blnext[bin_g]) - t_g
    state_ev = jnp.zeros((nev,), jnp.int32).at[e_g].set(state_g)
    delta_ev = jnp.zeros((nev,), jnp.float32).at[e_g].set(delta_g)

    t_cat = jnp.concatenate([t_bd, ts])
    state_cat = jnp.concatenate([state_bd.reshape(-1), state_ev])
    delta_cat = jnp.concatenate([delta_bd.reshape(-1), delta_ev])
    k_arr = jnp.arange(BINS, dtype=jnp.int32)
    bstart = k_arr * P + e_cnt
    estart = (k_arr + 1) * P + e_cnt
    starts = jnp.stack([bstart, estart], axis=1).reshape(-1)
    starts_pad = jnp.full((128,), _TPAD, jnp.int32).at[:2 * BINS].set(starts)

    t_o, ev_o, s_o, d_o = _assemble_sc(t_cat, state_cat, delta_cat, starts_pad)
    return (t_o[:T_TOTAL], ev_o[:T_TOTAL].astype(bool), s_o[:T_TOTAL],
            d_o[:T_TOTAL])
